# Initial kernel scaffold; baseline (speedup 1.0000x reference)
import jax
import jax.numpy as jnp
from jax.experimental import pallas as pl


def kernel(segment_labels, features, seg_table, w1, b1, w2, b2, Wout, bout):
    raise NotImplementedError("write your pallas kernel here")


# speedup vs baseline: 5.0075x; 5.0075x over previous
import functools

import jax
import jax.numpy as jnp
from jax.experimental import pallas as pl
from jax.experimental.pallas import tpu as pltpu

B = 2
H = 512
W = 512
HW = H * W
FEAT_DIM = 96
EMBED_DIM = 64
MAX_SEG = 32

PIX_BLK = 4096


def _main_body(f_ref, l_ref, w1_ref, b1_ref, sums_ref, cnt_ref):
    t = pl.program_id(1)

    @pl.when(t == 0)
    def _init():
        sums_ref[...] = jnp.zeros_like(sums_ref)
        cnt_ref[...] = jnp.zeros_like(cnt_ref)

    x = f_ref[0]
    w1 = w1_ref[...]
    b1 = b1_ref[...]
    h = jax.lax.dot_general(w1, x, (((1,), (0,)), ((), ())),
                            preferred_element_type=jnp.float32)
    h = jnp.maximum(h + b1, 0.0)

    lab = l_ref[0]
    sid = jax.lax.broadcasted_iota(jnp.int32, (MAX_SEG, PIX_BLK), 0)
    oh = (lab == sid).astype(jnp.float32)

    psum = jax.lax.dot_general(h, oh, (((1,), (1,)), ((), ())),
                               preferred_element_type=jnp.float32)
    ones = jnp.ones((8, PIX_BLK), jnp.float32)
    pcnt = jax.lax.dot_general(ones, oh, (((1,), (1,)), ((), ())),
                               preferred_element_type=jnp.float32)
    sums_ref[0] += psum
    cnt_ref[0] += pcnt


def _epilogue_body(sums_ref, cnt_ref, table_ref, w2_ref, b2_ref, wout_ref,
                   bout_ref, out_ref):
    w2 = w2_ref[...]
    wout = wout_ref[...]
    wa = wout[:, :EMBED_DIM]
    wb = wout[:, EMBED_DIM:]
    b2 = b2_ref[...]
    bout = bout_ref[...]
    emb = table_ref[...][:MAX_SEG]

    hp = jnp.float32
    g = jax.lax.dot_general(w2, wa, (((0,), (1,)), ((), ())),
                            preferred_element_type=hp,
                            precision=jax.lax.Precision.HIGHEST)
    const = jax.lax.dot_general(wb, emb, (((1,), (1,)), ((), ())),
                                preferred_element_type=hp,
                                precision=jax.lax.Precision.HIGHEST)
    const = const + jax.lax.dot_general(
        wa, b2, (((1,), (0,)), ((), ())), preferred_element_type=hp,
        precision=jax.lax.Precision.HIGHEST) + bout

    jj = jax.lax.broadcasted_iota(jnp.int32, (MAX_SEG, MAX_SEG), 0)
    ii = jax.lax.broadcasted_iota(jnp.int32, (MAX_SEG, MAX_SEG), 1)
    tri = (jj <= ii).astype(jnp.float32)

    for b in range(B):
        sums_b = sums_ref[b]
        cnt = cnt_ref[b][0:1]
        present = (cnt > 0.5).astype(jnp.float32)
        recip = 1.0 / jnp.maximum(cnt, 1.0)

        acc = jax.lax.dot_general(g, sums_b, (((0,), (0,)), ((), ())),
                                  preferred_element_type=hp,
                                  precision=jax.lax.Precision.HIGHEST)
        rows = acc * recip + const

        pos = jax.lax.dot_general(present, tri, (((1,), (0,)), ((), ())),
                                  preferred_element_type=hp,
                                  precision=jax.lax.Precision.HIGHEST)
        pos_i = pos.astype(jnp.int32) - 1
        dd = jax.lax.broadcasted_iota(jnp.int32, (MAX_SEG, MAX_SEG), 0)
        perm = ((dd == pos_i) & (present > 0.5)).astype(jnp.float32)

        out_b = jax.lax.dot_general(perm, rows, (((1,), (1,)), ((), ())),
                                    preferred_element_type=hp,
                                    precision=jax.lax.Precision.HIGHEST)
        out_ref[b] = out_b


def kernel(segment_labels, features, seg_table, w1, b1, w2, b2, Wout, bout):
    feats = features.reshape(B, FEAT_DIM, HW)
    labels = segment_labels.reshape(B, 1, HW)
    b1c = b1.reshape(EMBED_DIM, 1)
    b2c = b2.reshape(EMBED_DIM, 1)
    boutc = bout.reshape(EMBED_DIM, 1)

    grid = (B, HW // PIX_BLK)
    sums, cnts = pl.pallas_call(
        _main_body,
        grid=grid,
        in_specs=[
            pl.BlockSpec((1, FEAT_DIM, PIX_BLK), lambda b, t: (b, 0, t)),
            pl.BlockSpec((1, 1, PIX_BLK), lambda b, t: (b, 0, t)),
            pl.BlockSpec((EMBED_DIM, FEAT_DIM), lambda b, t: (0, 0)),
            pl.BlockSpec((EMBED_DIM, 1), lambda b, t: (0, 0)),
        ],
        out_specs=[
            pl.BlockSpec((1, EMBED_DIM, MAX_SEG), lambda b, t: (b, 0, 0)),
            pl.BlockSpec((1, 8, MAX_SEG), lambda b, t: (b, 0, 0)),
        ],
        out_shape=[
            jax.ShapeDtypeStruct((B, EMBED_DIM, MAX_SEG), jnp.float32),
            jax.ShapeDtypeStruct((B, 8, MAX_SEG), jnp.float32),
        ],
        compiler_params=pltpu.CompilerParams(
            dimension_semantics=("arbitrary", "arbitrary")),
    )(feats, labels, w1, b1c)

    out = pl.pallas_call(
        _epilogue_body,
        out_shape=jax.ShapeDtypeStruct((B, MAX_SEG, EMBED_DIM), jnp.float32),
    )(sums, cnts, seg_table, w2, b2c, Wout, boutc)
    return out


# speedup vs baseline: 5.8219x; 1.1626x over previous
import functools

import jax
import jax.numpy as jnp
from jax.experimental import pallas as pl
from jax.experimental.pallas import tpu as pltpu

B = 2
H = 512
W = 512
HW = H * W
FEAT_DIM = 96
EMBED_DIM = 64
MAX_SEG = 32

PIX_BLK = 16384


def _main_body(f_ref, l_ref, w1_ref, b1_ref, sums_ref, cnt_ref):
    t = pl.program_id(1)

    @pl.when(t == 0)
    def _init():
        sums_ref[...] = jnp.zeros_like(sums_ref)
        cnt_ref[...] = jnp.zeros_like(cnt_ref)

    x = f_ref[0]
    w1 = w1_ref[...]
    b1 = b1_ref[...]
    h = jax.lax.dot_general(w1, x, (((1,), (0,)), ((), ())),
                            preferred_element_type=jnp.float32)
    h = jnp.maximum(h + b1, 0.0)

    lab = l_ref[0]
    sid = jax.lax.broadcasted_iota(jnp.int32, (MAX_SEG, PIX_BLK), 0)
    oh = (lab == sid).astype(jnp.float32)

    psum = jax.lax.dot_general(h, oh, (((1,), (1,)), ((), ())),
                               preferred_element_type=jnp.float32)
    ones = jnp.ones((8, PIX_BLK), jnp.float32)
    pcnt = jax.lax.dot_general(ones, oh, (((1,), (1,)), ((), ())),
                               preferred_element_type=jnp.float32)
    sums_ref[0] += psum
    cnt_ref[0] += pcnt


def _epilogue_body(sums_ref, cnt_ref, table_ref, w2_ref, b2_ref, wout_ref,
                   bout_ref, out_ref):
    w2 = w2_ref[...]
    wout = wout_ref[...]
    wa = wout[:, :EMBED_DIM]
    wb = wout[:, EMBED_DIM:]
    b2 = b2_ref[...]
    bout = bout_ref[...]
    emb = table_ref[...][:MAX_SEG]

    hp = jnp.float32
    g = jax.lax.dot_general(w2, wa, (((0,), (1,)), ((), ())),
                            preferred_element_type=hp,
                            precision=jax.lax.Precision.HIGHEST)
    const = jax.lax.dot_general(wb, emb, (((1,), (1,)), ((), ())),
                                preferred_element_type=hp,
                                precision=jax.lax.Precision.HIGHEST)
    const = const + jax.lax.dot_general(
        wa, b2, (((1,), (0,)), ((), ())), preferred_element_type=hp,
        precision=jax.lax.Precision.HIGHEST) + bout

    jj = jax.lax.broadcasted_iota(jnp.int32, (MAX_SEG, MAX_SEG), 0)
    ii = jax.lax.broadcasted_iota(jnp.int32, (MAX_SEG, MAX_SEG), 1)
    tri = (jj <= ii).astype(jnp.float32)

    for b in range(B):
        sums_b = sums_ref[b]
        cnt = cnt_ref[b][0:1]
        present = (cnt > 0.5).astype(jnp.float32)
        recip = 1.0 / jnp.maximum(cnt, 1.0)

        acc = jax.lax.dot_general(g, sums_b, (((0,), (0,)), ((), ())),
                                  preferred_element_type=hp,
                                  precision=jax.lax.Precision.HIGHEST)
        rows = acc * recip + const

        pos = jax.lax.dot_general(present, tri, (((1,), (0,)), ((), ())),
                                  preferred_element_type=hp,
                                  precision=jax.lax.Precision.HIGHEST)
        pos_i = pos.astype(jnp.int32) - 1
        dd = jax.lax.broadcasted_iota(jnp.int32, (MAX_SEG, MAX_SEG), 0)
        perm = ((dd == pos_i) & (present > 0.5)).astype(jnp.float32)

        out_b = jax.lax.dot_general(perm, rows, (((1,), (1,)), ((), ())),
                                    preferred_element_type=hp,
                                    precision=jax.lax.Precision.HIGHEST)
        out_ref[b] = out_b


def kernel(segment_labels, features, seg_table, w1, b1, w2, b2, Wout, bout):
    feats = features.reshape(B, FEAT_DIM, HW)
    labels = segment_labels.reshape(B, 1, HW)
    b1c = b1.reshape(EMBED_DIM, 1)
    b2c = b2.reshape(EMBED_DIM, 1)
    boutc = bout.reshape(EMBED_DIM, 1)

    grid = (B, HW // PIX_BLK)
    sums, cnts = pl.pallas_call(
        _main_body,
        grid=grid,
        in_specs=[
            pl.BlockSpec((1, FEAT_DIM, PIX_BLK), lambda b, t: (b, 0, t)),
            pl.BlockSpec((1, 1, PIX_BLK), lambda b, t: (b, 0, t)),
            pl.BlockSpec((EMBED_DIM, FEAT_DIM), lambda b, t: (0, 0)),
            pl.BlockSpec((EMBED_DIM, 1), lambda b, t: (0, 0)),
        ],
        out_specs=[
            pl.BlockSpec((1, EMBED_DIM, MAX_SEG), lambda b, t: (b, 0, 0)),
            pl.BlockSpec((1, 8, MAX_SEG), lambda b, t: (b, 0, 0)),
        ],
        out_shape=[
            jax.ShapeDtypeStruct((B, EMBED_DIM, MAX_SEG), jnp.float32),
            jax.ShapeDtypeStruct((B, 8, MAX_SEG), jnp.float32),
        ],
        compiler_params=pltpu.CompilerParams(
            dimension_semantics=("arbitrary", "arbitrary")),
    )(feats, labels, w1, b1c)

    out = pl.pallas_call(
        _epilogue_body,
        out_shape=jax.ShapeDtypeStruct((B, MAX_SEG, EMBED_DIM), jnp.float32),
    )(sums, cnts, seg_table, w2, b2c, Wout, boutc)
    return out


# speedup vs baseline: 5.9437x; 1.0209x over previous
import functools

import jax
import jax.numpy as jnp
from jax.experimental import pallas as pl
from jax.experimental.pallas import tpu as pltpu

B = 2
H = 512
W = 512
HW = H * W
FEAT_DIM = 96
EMBED_DIM = 64
MAX_SEG = 32

PIX_BLK = 32768


def _main_body(f_ref, l_ref, w1_ref, b1_ref, sums_ref, cnt_ref):
    t = pl.program_id(1)

    @pl.when(t == 0)
    def _init():
        sums_ref[...] = jnp.zeros_like(sums_ref)
        cnt_ref[...] = jnp.zeros_like(cnt_ref)

    x = f_ref[0]
    w1 = w1_ref[...]
    b1 = b1_ref[...]
    h = jax.lax.dot_general(w1, x, (((1,), (0,)), ((), ())),
                            preferred_element_type=jnp.float32)
    h = jnp.maximum(h + b1, 0.0)

    lab = l_ref[0]
    sid = jax.lax.broadcasted_iota(jnp.int32, (MAX_SEG, PIX_BLK), 0)
    oh = (lab == sid).astype(jnp.float32)

    psum = jax.lax.dot_general(h, oh, (((1,), (1,)), ((), ())),
                               preferred_element_type=jnp.float32)
    ones = jnp.ones((8, PIX_BLK), jnp.float32)
    pcnt = jax.lax.dot_general(ones, oh, (((1,), (1,)), ((), ())),
                               preferred_element_type=jnp.float32)
    sums_ref[0] += psum
    cnt_ref[0] += pcnt


def _epilogue_body(sums_ref, cnt_ref, table_ref, w2_ref, b2_ref, wout_ref,
                   bout_ref, out_ref):
    w2 = w2_ref[...]
    wout = wout_ref[...]
    wa = wout[:, :EMBED_DIM]
    wb = wout[:, EMBED_DIM:]
    b2 = b2_ref[...]
    bout = bout_ref[...]
    emb = table_ref[...][:MAX_SEG]

    hp = jnp.float32
    g = jax.lax.dot_general(w2, wa, (((0,), (1,)), ((), ())),
                            preferred_element_type=hp,
                            precision=jax.lax.Precision.HIGHEST)
    const = jax.lax.dot_general(wb, emb, (((1,), (1,)), ((), ())),
                                preferred_element_type=hp,
                                precision=jax.lax.Precision.HIGHEST)
    const = const + jax.lax.dot_general(
        wa, b2, (((1,), (0,)), ((), ())), preferred_element_type=hp,
        precision=jax.lax.Precision.HIGHEST) + bout

    jj = jax.lax.broadcasted_iota(jnp.int32, (MAX_SEG, MAX_SEG), 0)
    ii = jax.lax.broadcasted_iota(jnp.int32, (MAX_SEG, MAX_SEG), 1)
    tri = (jj <= ii).astype(jnp.float32)

    for b in range(B):
        sums_b = sums_ref[b]
        cnt = cnt_ref[b][0:1]
        present = (cnt > 0.5).astype(jnp.float32)
        recip = 1.0 / jnp.maximum(cnt, 1.0)

        acc = jax.lax.dot_general(g, sums_b, (((0,), (0,)), ((), ())),
                                  preferred_element_type=hp,
                                  precision=jax.lax.Precision.HIGHEST)
        rows = acc * recip + const

        pos = jax.lax.dot_general(present, tri, (((1,), (0,)), ((), ())),
                                  preferred_element_type=hp,
                                  precision=jax.lax.Precision.HIGHEST)
        pos_i = pos.astype(jnp.int32) - 1
        dd = jax.lax.broadcasted_iota(jnp.int32, (MAX_SEG, MAX_SEG), 0)
        perm = ((dd == pos_i) & (present > 0.5)).astype(jnp.float32)

        out_b = jax.lax.dot_general(perm, rows, (((1,), (1,)), ((), ())),
                                    preferred_element_type=hp,
                                    precision=jax.lax.Precision.HIGHEST)
        out_ref[b] = out_b


def kernel(segment_labels, features, seg_table, w1, b1, w2, b2, Wout, bout):
    feats = features.reshape(B, FEAT_DIM, HW)
    labels = segment_labels.reshape(B, 1, HW)
    b1c = b1.reshape(EMBED_DIM, 1)
    b2c = b2.reshape(EMBED_DIM, 1)
    boutc = bout.reshape(EMBED_DIM, 1)

    grid = (B, HW // PIX_BLK)
    sums, cnts = pl.pallas_call(
        _main_body,
        grid=grid,
        in_specs=[
            pl.BlockSpec((1, FEAT_DIM, PIX_BLK), lambda b, t: (b, 0, t)),
            pl.BlockSpec((1, 1, PIX_BLK), lambda b, t: (b, 0, t)),
            pl.BlockSpec((EMBED_DIM, FEAT_DIM), lambda b, t: (0, 0)),
            pl.BlockSpec((EMBED_DIM, 1), lambda b, t: (0, 0)),
        ],
        out_specs=[
            pl.BlockSpec((1, EMBED_DIM, MAX_SEG), lambda b, t: (b, 0, 0)),
            pl.BlockSpec((1, 8, MAX_SEG), lambda b, t: (b, 0, 0)),
        ],
        out_shape=[
            jax.ShapeDtypeStruct((B, EMBED_DIM, MAX_SEG), jnp.float32),
            jax.ShapeDtypeStruct((B, 8, MAX_SEG), jnp.float32),
        ],
        compiler_params=pltpu.CompilerParams(
            dimension_semantics=("arbitrary", "arbitrary")),
    )(feats, labels, w1, b1c)

    out = pl.pallas_call(
        _epilogue_body,
        out_shape=jax.ShapeDtypeStruct((B, MAX_SEG, EMBED_DIM), jnp.float32),
    )(sums, cnts, seg_table, w2, b2c, Wout, boutc)
    return out


# speedup vs baseline: 5.9451x; 1.0002x over previous
import functools

import jax
import jax.numpy as jnp
from jax.experimental import pallas as pl
from jax.experimental.pallas import tpu as pltpu

B = 2
H = 512
W = 512
HW = H * W
FEAT_DIM = 96
EMBED_DIM = 64
MAX_SEG = 32

PIX_BLK = 32768


def _main_body(f_ref, l_ref, w1_ref, b1_ref, sums_ref, cnt_ref):
    t = pl.program_id(1)

    @pl.when(t == 0)
    def _init():
        sums_ref[...] = jnp.zeros_like(sums_ref)
        cnt_ref[...] = jnp.zeros_like(cnt_ref)

    x = f_ref[0].astype(jnp.bfloat16)
    w1 = w1_ref[...].astype(jnp.bfloat16)
    b1 = b1_ref[...]
    h = jax.lax.dot_general(w1, x, (((1,), (0,)), ((), ())),
                            preferred_element_type=jnp.float32)
    h = jnp.maximum(h + b1, 0.0).astype(jnp.bfloat16)

    lab = l_ref[0]
    sid = jax.lax.broadcasted_iota(jnp.int32, (MAX_SEG, PIX_BLK), 0)
    oh = (lab == sid).astype(jnp.bfloat16)

    psum = jax.lax.dot_general(h, oh, (((1,), (1,)), ((), ())),
                               preferred_element_type=jnp.float32)
    ones = jnp.ones((8, PIX_BLK), jnp.bfloat16)
    pcnt = jax.lax.dot_general(ones, oh, (((1,), (1,)), ((), ())),
                               preferred_element_type=jnp.float32)
    sums_ref[0] += psum
    cnt_ref[0] += pcnt


def _epilogue_body(sums_ref, cnt_ref, table_ref, w2_ref, b2_ref, wout_ref,
                   bout_ref, out_ref):
    w2 = w2_ref[...]
    wout = wout_ref[...]
    wa = wout[:, :EMBED_DIM]
    wb = wout[:, EMBED_DIM:]
    b2 = b2_ref[...]
    bout = bout_ref[...]
    emb = table_ref[...][:MAX_SEG]

    hp = jnp.float32
    g = jax.lax.dot_general(w2, wa, (((0,), (1,)), ((), ())),
                            preferred_element_type=hp,
                            precision=jax.lax.Precision.HIGHEST)
    const = jax.lax.dot_general(wb, emb, (((1,), (1,)), ((), ())),
                                preferred_element_type=hp,
                                precision=jax.lax.Precision.HIGHEST)
    const = const + jax.lax.dot_general(
        wa, b2, (((1,), (0,)), ((), ())), preferred_element_type=hp,
        precision=jax.lax.Precision.HIGHEST) + bout

    jj = jax.lax.broadcasted_iota(jnp.int32, (MAX_SEG, MAX_SEG), 0)
    ii = jax.lax.broadcasted_iota(jnp.int32, (MAX_SEG, MAX_SEG), 1)
    tri = (jj <= ii).astype(jnp.float32)

    for b in range(B):
        sums_b = sums_ref[b]
        cnt = cnt_ref[b][0:1]
        present = (cnt > 0.5).astype(jnp.float32)
        recip = 1.0 / jnp.maximum(cnt, 1.0)

        acc = jax.lax.dot_general(g, sums_b, (((0,), (0,)), ((), ())),
                                  preferred_element_type=hp,
                                  precision=jax.lax.Precision.HIGHEST)
        rows = acc * recip + const

        pos = jax.lax.dot_general(present, tri, (((1,), (0,)), ((), ())),
                                  preferred_element_type=hp,
                                  precision=jax.lax.Precision.HIGHEST)
        pos_i = pos.astype(jnp.int32) - 1
        dd = jax.lax.broadcasted_iota(jnp.int32, (MAX_SEG, MAX_SEG), 0)
        perm = ((dd == pos_i) & (present > 0.5)).astype(jnp.float32)

        out_b = jax.lax.dot_general(perm, rows, (((1,), (1,)), ((), ())),
                                    preferred_element_type=hp,
                                    precision=jax.lax.Precision.HIGHEST)
        out_ref[b] = out_b


def kernel(segment_labels, features, seg_table, w1, b1, w2, b2, Wout, bout):
    feats = features.reshape(B, FEAT_DIM, HW)
    labels = segment_labels.reshape(B, 1, HW)
    b1c = b1.reshape(EMBED_DIM, 1)
    b2c = b2.reshape(EMBED_DIM, 1)
    boutc = bout.reshape(EMBED_DIM, 1)

    grid = (B, HW // PIX_BLK)
    sums, cnts = pl.pallas_call(
        _main_body,
        grid=grid,
        in_specs=[
            pl.BlockSpec((1, FEAT_DIM, PIX_BLK), lambda b, t: (b, 0, t)),
            pl.BlockSpec((1, 1, PIX_BLK), lambda b, t: (b, 0, t)),
            pl.BlockSpec((EMBED_DIM, FEAT_DIM), lambda b, t: (0, 0)),
            pl.BlockSpec((EMBED_DIM, 1), lambda b, t: (0, 0)),
        ],
        out_specs=[
            pl.BlockSpec((1, EMBED_DIM, MAX_SEG), lambda b, t: (b, 0, 0)),
            pl.BlockSpec((1, 8, MAX_SEG), lambda b, t: (b, 0, 0)),
        ],
        out_shape=[
            jax.ShapeDtypeStruct((B, EMBED_DIM, MAX_SEG), jnp.float32),
            jax.ShapeDtypeStruct((B, 8, MAX_SEG), jnp.float32),
        ],
        compiler_params=pltpu.CompilerParams(
            dimension_semantics=("arbitrary", "arbitrary")),
    )(feats, labels, w1, b1c)

    out = pl.pallas_call(
        _epilogue_body,
        out_shape=jax.ShapeDtypeStruct((B, MAX_SEG, EMBED_DIM), jnp.float32),
    )(sums, cnts, seg_table, w2, b2c, Wout, boutc)
    return out
